# baseline (speedup 1.0000x reference)
import functools

import jax
import jax.numpy as jnp
from jax import lax
from jax.experimental import pallas as pl
from jax.experimental.pallas import tpu as pltpu
from jax.experimental.pallas import tpu_sc as plsc

IN_NODES = 2048
OUT_NODES = 32
BATCH = 32
F_SIZE = 16
BF = BATCH * F_SIZE
E = IN_NODES * OUT_NODES

L = 16
NK = BF // L
NW = 32
UPW = IN_NODES // NW
N_ITERS = 3

_GDN = lax.GatherDimensionNumbers(
    offset_dims=(), collapsed_slice_dims=(0,), start_index_map=(0,))


def _vperm(x, idx):
    return lax.gather(x, idx[:, None], _GDN, slice_sizes=(1,),
                      mode=lax.GatherScatterMode.PROMISE_IN_BOUNDS)


def _allsum(x, io):
    for sh in (8, 4, 2, 1):
        x = x + _vperm(x, io ^ sh)
    return x


def _allmax(x, io):
    for sh in (8, 4, 2, 1):
        x = jnp.maximum(x, _vperm(x, io ^ sh))
    return x


def _unpack_bf16(w):
    fe = plsc.bitcast(w << 16, jnp.float32)
    fo = plsc.bitcast(w & (-65536), jnp.float32)
    return fe, fo


def _make_sc_pass():
    mesh = plsc.VectorSubcoreMesh(core_axis_name="c", subcore_axis_name="s")

    @functools.partial(
        pl.kernel,
        mesh=mesh,
        compiler_params=pltpu.CompilerParams(needs_layout_passes=False),
        out_type=[
            jax.ShapeDtypeStruct((IN_NODES, OUT_NODES), jnp.float32),
            jax.ShapeDtypeStruct((NW, OUT_NODES, BF), jnp.float32),
        ],
        scratch_types=[
            pltpu.VMEM((OUT_NODES * BF // 2,), jnp.int32),
            pltpu.VMEM((OUT_NODES * BF // 2,), jnp.int32),
            pltpu.VMEM((OUT_NODES * BF // 2,), jnp.int32),
            pltpu.VMEM((OUT_NODES, BF), jnp.float32),
            pltpu.VMEM((UPW, OUT_NODES), jnp.float32),
            pltpu.SemaphoreType.DMA,
            pltpu.SemaphoreType.DMA,
        ],
    )
    def sc_pass(u_hbm, b_hbm, v_hbm, b_out, s_out, ubuf0, ubuf1, v_vm, s_vm,
                b_vm, sem0, sem1):
        wid = lax.axis_index("s") * 2 + lax.axis_index("c")
        u0 = wid * UPW
        io = lax.iota(jnp.int32, L)

        pltpu.sync_copy(v_hbm, v_vm)
        pltpu.sync_copy(b_hbm.at[pl.ds(u0, UPW)], b_vm)

        def zero_o(o, carry):
            for k in range(NK):
                s_vm[o, pl.ds(k * L, L)] = jnp.zeros((L,), jnp.float32)
            return carry

        lax.fori_loop(0, OUT_NODES, zero_o, 0)

        def compute(slot: int, u_local):
            ub = ubuf0 if slot == 0 else ubuf1

            def dot_o(o, carry):
                d0, d1 = carry
                acc0 = jnp.zeros((L,), jnp.float32)
                acc1 = jnp.zeros((L,), jnp.float32)
                ob = o * (BF // 2)
                for k2 in range(NK // 2):
                    sl = pl.ds(ob + L * k2, L)
                    ue, uo = _unpack_bf16(ub[sl])
                    ve, vo = _unpack_bf16(v_vm[sl])
                    acc0 = acc0 + ue * ve
                    acc1 = acc1 + uo * vo
                t = _allsum(acc0 + acc1, io) * (1.0 / BATCH)
                d0 = jnp.where(io == o, d0 + t, d0)
                d1 = jnp.where(io == o - L, d1 + t, d1)
                return d0, d1

            z = jnp.zeros((L,), jnp.float32)
            d0, d1 = lax.fori_loop(0, OUT_NODES, dot_o, (z, z))
            b0 = b_vm[u_local, pl.ds(0, L)] + d0
            b1 = b_vm[u_local, pl.ds(L, L)] + d1
            b_vm[u_local, pl.ds(0, L)] = b0
            b_vm[u_local, pl.ds(L, L)] = b1

            m = _allmax(jnp.maximum(b0, b1), io)
            e0 = jnp.exp(b0 - m)
            e1 = jnp.exp(b1 - m)
            denom = _allsum(e0 + e1, io)
            c0 = e0 / denom
            c1 = e1 / denom

            def accum_o(o, carry):
                so = jnp.full((L,), 0, jnp.int32) + (o & (L - 1))
                cs = jnp.where(o < L, _vperm(c0, so), _vperm(c1, so))
                ob = o * (BF // 2)
                for k2 in range(NK // 2):
                    ue, uo = _unpack_bf16(ub[pl.ds(ob + L * k2, L)])
                    sle = pl.ds(2 * L * k2, L)
                    slo = pl.ds(2 * L * k2 + L, L)
                    s_vm[o, sle] = s_vm[o, sle] + cs * ue
                    s_vm[o, slo] = s_vm[o, slo] + cs * uo
                return carry

            lax.fori_loop(0, OUT_NODES, accum_o, 0)

        pltpu.make_async_copy(u_hbm.at[pl.ds(u0 * (OUT_NODES * BF // 2), OUT_NODES * BF // 2)], ubuf0, sem0).start()

        def pair(i, carry):
            u = 2 * i
            pltpu.make_async_copy(u_hbm.at[pl.ds((u0 + u + 1) * (OUT_NODES * BF // 2), OUT_NODES * BF // 2)], ubuf1, sem1).start()
            pltpu.make_async_copy(u_hbm.at[pl.ds((u0 + u) * (OUT_NODES * BF // 2), OUT_NODES * BF // 2)], ubuf0, sem0).wait()
            compute(0, u)

            @pl.when(u + 2 < UPW)
            def _():
                pltpu.make_async_copy(
                    u_hbm.at[pl.ds((u0 + u + 2) * (OUT_NODES * BF // 2),
                                   OUT_NODES * BF // 2)],
                    ubuf0, sem0).start()

            pltpu.make_async_copy(u_hbm.at[pl.ds((u0 + u + 1) * (OUT_NODES * BF // 2), OUT_NODES * BF // 2)], ubuf1, sem1).wait()
            compute(1, u + 1)
            return carry

        lax.fori_loop(0, UPW // 2, pair, 0)

        pltpu.sync_copy(b_vm, b_out.at[pl.ds(u0, UPW)])
        pltpu.sync_copy(s_vm, s_out.at[wid])

    return sc_pass


_sc_pass = _make_sc_pass()


def _squash_body(sp_ref, v_ref, vbf_ref):
    sp = jnp.sum(sp_ref[...], axis=0)
    s = sp.reshape(OUT_NODES, NK // 2, 2, L).transpose(0, 1, 3, 2)
    s = s.reshape(OUT_NODES, BF)
    ss = s * s
    r = lax.broadcasted_iota(jnp.int32, (BF, BATCH), 0)
    g = lax.broadcasted_iota(jnp.int32, (BF, BATCH), 1)
    m1 = (r // F_SIZE == g).astype(jnp.float32)
    grp = jnp.dot(ss, m1, preferred_element_type=jnp.float32)
    sq = jnp.dot(grp, m1.T, preferred_element_type=jnp.float32)
    norm = jnp.sqrt(sq)
    v = s * (sq / ((1.0 + sq) * norm))
    v_ref[...] = v
    vbf_ref[...] = v.astype(jnp.bfloat16)


def _squash(s_part):
    return pl.pallas_call(
        _squash_body,
        out_shape=[
            jax.ShapeDtypeStruct((OUT_NODES, BF), jnp.float32),
            jax.ShapeDtypeStruct((OUT_NODES, BF), jnp.bfloat16),
        ],
    )(s_part)


@jax.jit
def _routing(u_bf, b2):
    v_bf = jnp.zeros((OUT_NODES * BF // 2,), jnp.int32)
    b_cur = b2
    v = None
    for _ in range(N_ITERS):
        b_cur, s_part = _sc_pass(u_bf, b_cur, v_bf)
        v, v_bf2 = _squash(s_part)
        v_bf = lax.bitcast_convert_type(
            v_bf2.reshape(OUT_NODES * BF // 2, 2), jnp.int32)
    return v


def kernel(u_hat, b, routing_num):
    del routing_num
    u_bf = lax.bitcast_convert_type(
        u_hat.reshape(E * BF // 2, 2).astype(jnp.bfloat16), jnp.int32)
    b2 = b.reshape(IN_NODES, OUT_NODES)
    v = _routing(u_bf, b2)
    return v.reshape(OUT_NODES, BATCH, F_SIZE)


# speedup vs baseline: 3.7552x; 3.7552x over previous
import functools

import jax
import jax.numpy as jnp
from jax import lax
from jax.experimental import pallas as pl
from jax.experimental.pallas import tpu as pltpu
from jax.experimental.pallas import tpu_sc as plsc

IN_NODES = 2048
OUT_NODES = 32
BATCH = 32
F_SIZE = 16
BF = BATCH * F_SIZE
E = IN_NODES * OUT_NODES

L = 16
NK = BF // L
NW = 32
UPW = IN_NODES // NW
N_ITERS = 3

_GDN = lax.GatherDimensionNumbers(
    offset_dims=(), collapsed_slice_dims=(0,), start_index_map=(0,))


def _vperm(x, idx):
    return lax.gather(x, idx[:, None], _GDN, slice_sizes=(1,),
                      mode=lax.GatherScatterMode.PROMISE_IN_BOUNDS)


def _allsum(x, io):
    for sh in (8, 4, 2, 1):
        x = x + _vperm(x, io ^ sh)
    return x


def _allmax(x, io):
    for sh in (8, 4, 2, 1):
        x = jnp.maximum(x, _vperm(x, io ^ sh))
    return x


def _make_sc_pass():
    mesh = plsc.VectorSubcoreMesh(core_axis_name="c", subcore_axis_name="s")

    @functools.partial(
        pl.kernel,
        mesh=mesh,
        out_type=[
            jax.ShapeDtypeStruct((IN_NODES, OUT_NODES), jnp.float32),
            jax.ShapeDtypeStruct((NW, OUT_NODES, BF), jnp.float32),
        ],
        scratch_types=[
            pltpu.VMEM((2, 4, 4, 8, 128), jnp.float32),
            pltpu.VMEM((OUT_NODES, BF), jnp.float32),
            pltpu.VMEM((OUT_NODES, BF), jnp.float32),
            pltpu.VMEM((UPW, OUT_NODES), jnp.float32),
            pltpu.SemaphoreType.DMA,
            pltpu.SemaphoreType.DMA,
        ],
    )
    def sc_pass(u_hbm, b_hbm, v_hbm, b_out, s_out, ubuf, v_vm, s_vm, b_vm,
                sem0, sem1):
        wid = lax.axis_index("s") * 2 + lax.axis_index("c")
        u0 = wid * UPW
        io = lax.iota(jnp.int32, L)

        pltpu.sync_copy(v_hbm, v_vm)
        pltpu.sync_copy(b_hbm.at[pl.ds(u0, UPW)], b_vm)

        def zero_o(o, carry):
            for k in range(NK):
                s_vm[o, pl.ds(k * L, L)] = jnp.zeros((L,), jnp.float32)
            return carry

        lax.fori_loop(0, OUT_NODES, zero_o, 0)

        def compute(slot: int, u_local):
            ub = ubuf.at[slot]

            def dot_o(o, carry):
                d0, d1 = carry
                acc0 = jnp.zeros((L,), jnp.float32)
                acc1 = jnp.zeros((L,), jnp.float32)
                tr = o // 8
                r = o - 8 * tr
                for k in range(0, NK, 2):
                    sl0 = pl.ds(k * L, L)
                    sl1 = pl.ds((k + 1) * L, L)
                    u0c = ub[tr, k // 8, r, pl.ds(k % 8 * L, L)]
                    u1c = ub[tr, (k + 1) // 8, r, pl.ds((k + 1) % 8 * L, L)]
                    acc0 = acc0 + u0c * v_vm[o, sl0]
                    acc1 = acc1 + u1c * v_vm[o, sl1]
                t = _allsum(acc0 + acc1, io) * (1.0 / BATCH)
                d0 = jnp.where(io == o, d0 + t, d0)
                d1 = jnp.where(io == o - L, d1 + t, d1)
                return d0, d1

            z = jnp.zeros((L,), jnp.float32)
            d0, d1 = lax.fori_loop(0, OUT_NODES, dot_o, (z, z))
            b0 = b_vm[u_local, pl.ds(0, L)] + d0
            b1 = b_vm[u_local, pl.ds(L, L)] + d1
            b_vm[u_local, pl.ds(0, L)] = b0
            b_vm[u_local, pl.ds(L, L)] = b1

            m = _allmax(jnp.maximum(b0, b1), io)
            e0 = jnp.exp(b0 - m)
            e1 = jnp.exp(b1 - m)
            denom = _allsum(e0 + e1, io)
            c0 = e0 / denom
            c1 = e1 / denom

            def accum_o(o, carry):
                so = jnp.full((L,), 0, jnp.int32) + (o & (L - 1))
                cs = jnp.where(o < L, _vperm(c0, so), _vperm(c1, so))
                tr = o // 8
                r = o - 8 * tr
                for k in range(NK):
                    sl = pl.ds(k * L, L)
                    uc = ub[tr, k // 8, r, pl.ds(k % 8 * L, L)]
                    s_vm[o, sl] = s_vm[o, sl] + cs * uc
                return carry

            lax.fori_loop(0, OUT_NODES, accum_o, 0)

        pltpu.make_async_copy(u_hbm.at[pl.ds(u0 * 4, 4)], ubuf.at[0], sem0).start()

        def pair(i, carry):
            u = 2 * i
            pltpu.make_async_copy(u_hbm.at[pl.ds((u0 + u + 1) * 4, 4)], ubuf.at[1], sem1).start()
            pltpu.make_async_copy(u_hbm.at[pl.ds((u0 + u) * 4, 4)], ubuf.at[0], sem0).wait()
            compute(0, u)

            @pl.when(u + 2 < UPW)
            def _():
                pltpu.make_async_copy(
                    u_hbm.at[pl.ds((u0 + u + 2) * 4, 4)],
                    ubuf.at[0], sem0).start()

            pltpu.make_async_copy(u_hbm.at[pl.ds((u0 + u + 1) * OUT_NODES, OUT_NODES)], ubuf.at[1], sem1).wait()
            compute(1, u + 1)
            return carry

        lax.fori_loop(0, UPW // 2, pair, 0)

        pltpu.sync_copy(b_vm, b_out.at[pl.ds(u0, UPW)])
        pltpu.sync_copy(s_vm, s_out.at[wid])

    return sc_pass


_sc_pass = _make_sc_pass()


def _squash_body(sp_ref, v_ref):
    s = jnp.sum(sp_ref[...], axis=0)
    ss = s * s
    r = lax.broadcasted_iota(jnp.int32, (BF, BATCH), 0)
    g = lax.broadcasted_iota(jnp.int32, (BF, BATCH), 1)
    m1 = (r // F_SIZE == g).astype(jnp.float32)
    grp = jnp.dot(ss, m1, preferred_element_type=jnp.float32)
    sq = jnp.dot(grp, m1.T, preferred_element_type=jnp.float32)
    norm = jnp.sqrt(sq)
    v_ref[...] = s * (sq / ((1.0 + sq) * norm))


def _squash(s_part):
    return pl.pallas_call(
        _squash_body,
        out_shape=jax.ShapeDtypeStruct((OUT_NODES, BF), jnp.float32),
    )(s_part)


@jax.jit
def _routing(u3, b2):
    v = jnp.zeros((OUT_NODES, BF), jnp.float32)
    b_cur = b2
    for _ in range(N_ITERS):
        b_cur, s_part = _sc_pass(u3, b_cur, v)
        v = _squash(s_part)
    return v


def kernel(u_hat, b, routing_num):
    del routing_num
    u_tiles = (u_hat.reshape(E // 8, 8, 4, 128).transpose(0, 2, 1, 3))
    b2 = b.reshape(IN_NODES, OUT_NODES)
    v = _routing(u_tiles, b2)
    return v.reshape(OUT_NODES, BATCH, F_SIZE)


# speedup vs baseline: 14.0592x; 3.7439x over previous
import functools

import jax
import jax.numpy as jnp
from jax import lax
from jax.experimental import pallas as pl
from jax.experimental.pallas import tpu as pltpu
from jax.experimental.pallas import tpu_sc as plsc

IN_NODES = 2048
OUT_NODES = 32
BATCH = 32
F_SIZE = 16
BF = BATCH * F_SIZE
E = IN_NODES * OUT_NODES

L = 16
NK = BF // L
NW = 32
UPW = IN_NODES // NW
N_ITERS = 3

_GDN = lax.GatherDimensionNumbers(
    offset_dims=(), collapsed_slice_dims=(0,), start_index_map=(0,))


def _vperm(x, idx):
    return lax.gather(x, idx[:, None], _GDN, slice_sizes=(1,),
                      mode=lax.GatherScatterMode.PROMISE_IN_BOUNDS)


def _allsum(x, io):
    for sh in (8, 4, 2, 1):
        x = x + _vperm(x, io ^ sh)
    return x


def _allmax(x, io):
    for sh in (8, 4, 2, 1):
        x = jnp.maximum(x, _vperm(x, io ^ sh))
    return x


def _make_sc_pass():
    mesh = plsc.VectorSubcoreMesh(core_axis_name="c", subcore_axis_name="s")

    @functools.partial(
        pl.kernel,
        mesh=mesh,
        out_type=[
            jax.ShapeDtypeStruct((IN_NODES, OUT_NODES), jnp.float32),
            jax.ShapeDtypeStruct((NW, OUT_NODES, BF), jnp.float32),
        ],
        scratch_types=[
            pltpu.VMEM((4, OUT_NODES, BF), jnp.float32),
            pltpu.VMEM((OUT_NODES, BF), jnp.float32),
            pltpu.VMEM((OUT_NODES, BF), jnp.float32),
            pltpu.VMEM((UPW, OUT_NODES), jnp.float32),
            pltpu.SemaphoreType.DMA,
            pltpu.SemaphoreType.DMA,
            pltpu.SemaphoreType.DMA,
            pltpu.SemaphoreType.DMA,
        ],
    )
    def sc_pass(u_hbm, b_hbm, v_hbm, b_out, s_out, ubuf, v_vm, s_vm, b_vm,
                sem0, sem1, sem2, sem3):
        wid = lax.axis_index("s") * 2 + lax.axis_index("c")
        u0 = wid * UPW
        io = lax.iota(jnp.int32, L)

        pltpu.sync_copy(v_hbm, v_vm)
        pltpu.sync_copy(b_hbm.at[pl.ds(u0, UPW)], b_vm)

        def zero_o(o, carry):
            for k in range(NK):
                s_vm[o, pl.ds(k * L, L)] = jnp.zeros((L,), jnp.float32)
            return carry

        lax.fori_loop(0, OUT_NODES, zero_o, 0)

        def phase_a(slot: int, u_local):
            ub = ubuf.at[slot]

            def dot_o(o, carry):
                d0, d1 = carry
                acc0 = jnp.zeros((L,), jnp.float32)
                acc1 = jnp.zeros((L,), jnp.float32)
                for k in range(0, NK, 2):
                    sl0 = pl.ds(k * L, L)
                    sl1 = pl.ds((k + 1) * L, L)
                    acc0 = acc0 + ub[o, sl0] * v_vm[o, sl0]
                    acc1 = acc1 + ub[o, sl1] * v_vm[o, sl1]
                t = _allsum(acc0 + acc1, io) * (1.0 / BATCH)
                d0 = jnp.where(io == o, d0 + t, d0)
                d1 = jnp.where(io == o - L, d1 + t, d1)
                return d0, d1

            z = jnp.zeros((L,), jnp.float32)
            d0, d1 = lax.fori_loop(0, OUT_NODES, dot_o, (z, z))
            b0 = b_vm[u_local, pl.ds(0, L)] + d0
            b1 = b_vm[u_local, pl.ds(L, L)] + d1
            b_vm[u_local, pl.ds(0, L)] = b0
            b_vm[u_local, pl.ds(L, L)] = b1

            m = _allmax(jnp.maximum(b0, b1), io)
            e0 = jnp.exp(b0 - m)
            e1 = jnp.exp(b1 - m)
            denom = _allsum(e0 + e1, io)
            return e0 / denom, e1 / denom

        def accum_pair(slot_a: int, slot_b: int, ca, cb):
            ua = ubuf.at[slot_a]
            ub2 = ubuf.at[slot_b]
            ca0, ca1 = ca
            cb0, cb1 = cb

            def accum_o(o, carry):
                so = jnp.full((L,), 0, jnp.int32) + (o & (L - 1))
                csa = jnp.where(o < L, _vperm(ca0, so), _vperm(ca1, so))
                csb = jnp.where(o < L, _vperm(cb0, so), _vperm(cb1, so))
                for k in range(NK):
                    sl = pl.ds(k * L, L)
                    s_vm[o, sl] = (s_vm[o, sl] + csa * ua[o, sl]
                                   + csb * ub2[o, sl])
                return carry

            lax.fori_loop(0, OUT_NODES, accum_o, 0)

        sems = (sem0, sem1, sem2, sem3)

        def dma(node, slot: int):
            return pltpu.make_async_copy(
                u_hbm.at[pl.ds((u0 + node) * OUT_NODES, OUT_NODES)],
                ubuf.at[slot], sems[slot])

        dma(0, 0).start()
        dma(1, 1).start()

        def quad(i, carry):
            u = 4 * i

            @pl.when(u + 3 < UPW)
            def _():
                dma(u + 2, 2).start()
                dma(u + 3, 3).start()

            dma(u, 0).wait()
            ca = phase_a(0, u)
            dma(u + 1, 1).wait()
            cb = phase_a(1, u + 1)
            accum_pair(0, 1, ca, cb)

            @pl.when(u + 5 < UPW)
            def _():
                dma(u + 4, 0).start()
                dma(u + 5, 1).start()

            dma(u + 2, 2).wait()
            cc = phase_a(2, u + 2)
            dma(u + 3, 3).wait()
            cd = phase_a(3, u + 3)
            accum_pair(2, 3, cc, cd)
            return carry

        lax.fori_loop(0, UPW // 4, quad, 0)

        pltpu.sync_copy(b_vm, b_out.at[pl.ds(u0, UPW)])
        pltpu.sync_copy(s_vm, s_out.at[wid])

    return sc_pass


_sc_pass = _make_sc_pass()


def _squash_body(sp_ref, v_ref):
    s = jnp.sum(sp_ref[...], axis=0)
    ss = s * s
    r = lax.broadcasted_iota(jnp.int32, (BF, BATCH), 0)
    g = lax.broadcasted_iota(jnp.int32, (BF, BATCH), 1)
    m1 = (r // F_SIZE == g).astype(jnp.float32)
    grp = jnp.dot(ss, m1, preferred_element_type=jnp.float32)
    sq = jnp.dot(grp, m1.T, preferred_element_type=jnp.float32)
    norm = jnp.sqrt(sq)
    v_ref[...] = s * (sq / ((1.0 + sq) * norm))


def _squash(s_part):
    return pl.pallas_call(
        _squash_body,
        out_shape=jax.ShapeDtypeStruct((OUT_NODES, BF), jnp.float32),
    )(s_part)


@jax.jit
def _routing(u3, b2):
    v = jnp.zeros((OUT_NODES, BF), jnp.float32)
    b_cur = b2
    for _ in range(N_ITERS):
        b_cur, s_part = _sc_pass(u3, b_cur, v)
        v = _squash(s_part)
    return v


def kernel(u_hat, b, routing_num):
    del routing_num
    u_flat = u_hat.reshape(E, BF)
    b2 = b.reshape(IN_NODES, OUT_NODES)
    v = _routing(u_flat, b2)
    return v.reshape(OUT_NODES, BATCH, F_SIZE)
